# Initial kernel scaffold; baseline (speedup 1.0000x reference)
import jax
import jax.numpy as jnp
from jax.experimental import pallas as pl


def kernel(x, adj, gate_adj, gc1_W, gc1_b, gc1_l1W, gc1_l1b, gc1_l2W, gc1_l2b, gc2_W, gc2_b, gc2_l1W, gc2_l1b, gc2_l2W, gc2_l2b):
    raise NotImplementedError("write your pallas kernel here")


# speedup vs baseline: 1.0418x; 1.0418x over previous
import jax
import jax.numpy as jnp
from jax.experimental import pallas as pl

_N = 10000
_D = 128
_R = 200


def _prologue_kernel(x_ref, w1_ref, l1w1_ref, l2w1_ref, w2_ref, l1w2_ref,
                     l2w2_ref, ha_ref, hg_ref, wa2_ref, wg2_ref):
    xb = x_ref[...].astype(jnp.bfloat16)
    h = jnp.dot(xb, w1_ref[...].astype(jnp.bfloat16),
                preferred_element_type=jnp.float32)
    hb = h.astype(jnp.bfloat16)
    pa = jnp.dot(hb, l1w1_ref[...].astype(jnp.bfloat16),
                 preferred_element_type=jnp.float32)
    pg = jnp.dot(hb, l2w1_ref[...].astype(jnp.bfloat16),
                 preferred_element_type=jnp.float32)
    ha_ref[...] = jnp.concatenate([h, pa], axis=1).astype(jnp.bfloat16)
    hg_ref[...] = jnp.concatenate([h, pg], axis=1).astype(jnp.bfloat16)
    w2b = w2_ref[...].astype(jnp.bfloat16)
    w2a = jnp.dot(w2b, l1w2_ref[...].astype(jnp.bfloat16),
                  preferred_element_type=jnp.float32)
    w2g = jnp.dot(w2b, l2w2_ref[...].astype(jnp.bfloat16),
                  preferred_element_type=jnp.float32)
    wa2_ref[...] = jnp.concatenate([w2_ref[...], w2a], axis=1).astype(jnp.bfloat16)
    wg2_ref[...] = jnp.concatenate([w2_ref[...], w2g], axis=1).astype(jnp.bfloat16)


def _layer1_kernel(adj_ref, gate_ref, ha_ref, hg_ref, wa2_ref, wg2_ref,
                   b1_ref, l1b_ref, l2b_ref, oa_ref, og_ref):
    a = adj_ref[...].astype(jnp.bfloat16)
    gm = gate_ref[...].astype(jnp.bfloat16)
    acc_a = jnp.dot(a, ha_ref[...], preferred_element_type=jnp.float32)
    acc_g = jnp.dot(gm, hg_ref[...], preferred_element_type=jnp.float32)
    s = acc_a[:, _D:] + acc_g[:, _D:] + l1b_ref[...] + l2b_ref[...]
    g = jax.nn.sigmoid(s)
    y = g * acc_a[:, :_D] + (1.0 - g) * acc_g[:, :_D] + b1_ref[...]
    yb = jnp.maximum(y, 0.0).astype(jnp.bfloat16)
    oa_ref[...] = jnp.dot(yb, wa2_ref[...],
                          preferred_element_type=jnp.float32).astype(jnp.bfloat16)
    og_ref[...] = jnp.dot(yb, wg2_ref[...],
                          preferred_element_type=jnp.float32).astype(jnp.bfloat16)


def _layer2_kernel(adj_ref, gate_ref, ha_ref, hg_ref, x_ref,
                   b2_ref, l1b_ref, l2b_ref, out_ref):
    a = adj_ref[...].astype(jnp.bfloat16)
    gm = gate_ref[...].astype(jnp.bfloat16)
    acc_a = jnp.dot(a, ha_ref[...], preferred_element_type=jnp.float32)
    acc_g = jnp.dot(gm, hg_ref[...], preferred_element_type=jnp.float32)
    s = acc_a[:, _D:] + acc_g[:, _D:] + l1b_ref[...] + l2b_ref[...]
    g = jax.nn.sigmoid(s)
    y = g * acc_a[:, :_D] + (1.0 - g) * acc_g[:, :_D] + b2_ref[...]
    out_ref[...] = jnp.maximum(y, 0.0) + x_ref[...]


def kernel(x, adj, gate_adj, gc1_W, gc1_b, gc1_l1W, gc1_l1b, gc1_l2W,
           gc1_l2b, gc2_W, gc2_b, gc2_l1W, gc2_l1b, gc2_l2W, gc2_l2b):
    f32 = jnp.float32
    bf16 = jnp.bfloat16

    ha1, hg1, wa2, wg2 = pl.pallas_call(
        _prologue_kernel,
        out_shape=[
            jax.ShapeDtypeStruct((_N, 2 * _D), bf16),
            jax.ShapeDtypeStruct((_N, 2 * _D), bf16),
            jax.ShapeDtypeStruct((_D, 2 * _D), bf16),
            jax.ShapeDtypeStruct((_D, 2 * _D), bf16),
        ],
    )(x, gc1_W, gc1_l1W, gc1_l2W, gc2_W, gc2_l1W, gc2_l2W)

    b1 = gc1_b.reshape(1, _D)
    l1b1 = gc1_l1b.reshape(1, _D)
    l2b1 = gc1_l2b.reshape(1, _D)
    b2 = gc2_b.reshape(1, _D)
    l1b2 = gc2_l1b.reshape(1, _D)
    l2b2 = gc2_l2b.reshape(1, _D)

    nblk = _N // _R
    row_spec = pl.BlockSpec((_R, _N), lambda i: (i, 0))
    full_spec = pl.BlockSpec((_N, 2 * _D), lambda i: (0, 0))
    w_spec = pl.BlockSpec((_D, 2 * _D), lambda i: (0, 0))
    bias_spec = pl.BlockSpec((1, _D), lambda i: (0, 0))
    out_cat_spec = pl.BlockSpec((_R, 2 * _D), lambda i: (i, 0))
    out_d_spec = pl.BlockSpec((_R, _D), lambda i: (i, 0))

    ha2, hg2 = pl.pallas_call(
        _layer1_kernel,
        grid=(nblk,),
        in_specs=[row_spec, row_spec, full_spec, full_spec, w_spec, w_spec,
                  bias_spec, bias_spec, bias_spec],
        out_specs=[out_cat_spec, out_cat_spec],
        out_shape=[
            jax.ShapeDtypeStruct((_N, 2 * _D), bf16),
            jax.ShapeDtypeStruct((_N, 2 * _D), bf16),
        ],
    )(adj, gate_adj, ha1, hg1, wa2, wg2, b1, l1b1, l2b1)

    out = pl.pallas_call(
        _layer2_kernel,
        grid=(nblk,),
        in_specs=[row_spec, row_spec, full_spec, full_spec, out_d_spec,
                  bias_spec, bias_spec, bias_spec],
        out_specs=out_d_spec,
        out_shape=jax.ShapeDtypeStruct((_N, _D), f32),
    )(adj, gate_adj, ha2, hg2, x, b2, l1b2, l2b2)

    return out
